# Initial kernel scaffold; baseline (speedup 1.0000x reference)
import functools

import jax
import jax.numpy as jnp
from jax import lax
from jax.experimental import pallas as pl
from jax.experimental.pallas import tpu as pltpu
from jax.experimental.pallas import tpu_sc as plsc

E = 8
D = 1024
F = 4096
T = 2048
C = 384
SLOTS = E * C
SCRATCH = SLOTS

RR = 256
WW = 8

NC = 2
NS = 16
NW = NC * NS
TPW = T // NW
LANES = 16

FB = 1024
NF = F // FB


def _route_body(eids_ref, g_ref):
    eids = eids_ref[...]
    a = lax.broadcasted_iota(jnp.float32, (RR, RR), 0)
    b = lax.broadcasted_iota(jnp.float32, (RR, RR), 1)
    ltri = (b < a).astype(jnp.float32)

    def incl_cumsum_lanes(m):
        c = m
        for k in (1, 2, 4):
            z = jnp.zeros((RR, k), jnp.float32)
            c = c + jnp.concatenate([z, c[:, : WW - k]], axis=1)
        return c

    withins = []
    rowtots = []
    for e in range(E):
        m = (eids == e).astype(jnp.float32)
        w = incl_cumsum_lanes(m)
        withins.append(w)
        rowtots.append(w[:, WW - 1 : WW])
    rowtot = jnp.concatenate(rowtots, axis=1)
    rowpref = jnp.dot(ltri, rowtot, preferred_element_type=jnp.float32)

    rank = jnp.zeros((RR, WW), jnp.float32)
    for e in range(E):
        m = (eids == e).astype(jnp.float32)
        rank = rank + m * (withins[e] - 1.0 + rowpref[:, e : e + 1])
    rank_i = rank.astype(jnp.int32)
    g = jnp.where(rank_i < C, eids * C + rank_i, SCRATCH)
    g_ref[...] = g.astype(jnp.int32)


def _route(eids2d):
    return pl.pallas_call(
        _route_body,
        out_shape=jax.ShapeDtypeStruct((RR, WW), jnp.int32),
    )(eids2d)


_SC_MESH = plsc.VectorSubcoreMesh(core_axis_name="c", subcore_axis_name="s")


@functools.partial(
    pl.kernel,
    mesh=_SC_MESH,
    out_type=jax.ShapeDtypeStruct((SLOTS + 1, D), jnp.float32),
    scratch_types=[
        pltpu.VMEM((TPW,), jnp.int32),
        pltpu.VMEM((TPW, D), jnp.float32),
        pltpu.SemaphoreType.DMA,
    ],
)
def _scatter(tokens_hbm, g_hbm, padded_hbm, idx_v, rows_v, sem):
    wid = lax.axis_index("s") * NC + lax.axis_index("c")
    base = wid * TPW
    pltpu.sync_copy(g_hbm.at[pl.ds(base, TPW)], idx_v)
    pltpu.sync_copy(tokens_hbm.at[pl.ds(base, TPW)], rows_v)
    pltpu.async_copy(rows_v, padded_hbm.at[idx_v], sem).wait()


def _ffn_body(x_ref, w1_ref, w3_ref, w2_ref, o_ref):
    f = pl.program_id(1)
    x = x_ref[...].astype(jnp.bfloat16)
    gate = jnp.dot(x, w1_ref[0].astype(jnp.bfloat16),
                   preferred_element_type=jnp.float32)
    up = jnp.dot(x, w3_ref[0].astype(jnp.bfloat16),
                 preferred_element_type=jnp.float32)
    h = (gate * jax.nn.sigmoid(gate)) * up
    acc = jnp.dot(h.astype(jnp.bfloat16), w2_ref[0].astype(jnp.bfloat16),
                  preferred_element_type=jnp.float32)

    @pl.when(f == 0)
    def _():
        o_ref[...] = acc

    @pl.when(f != 0)
    def _():
        o_ref[...] = o_ref[...] + acc


def _ffn(padded, w1, w3, w2):
    return pl.pallas_call(
        _ffn_body,
        grid=(E, NF),
        in_specs=[
            pl.BlockSpec((C, D), lambda e, f: (e, 0)),
            pl.BlockSpec((1, D, FB), lambda e, f: (e, 0, f)),
            pl.BlockSpec((1, D, FB), lambda e, f: (e, 0, f)),
            pl.BlockSpec((1, FB, D), lambda e, f: (e, f, 0)),
        ],
        out_specs=pl.BlockSpec((C, D), lambda e, f: (e, 0)),
        out_shape=jax.ShapeDtypeStruct((SLOTS, D), jnp.float32),
    )(padded, w1, w3, w2)


@functools.partial(
    pl.kernel,
    mesh=_SC_MESH,
    out_type=jax.ShapeDtypeStruct((T, D), jnp.float32),
    scratch_types=[
        pltpu.VMEM((TPW,), jnp.int32),
        pltpu.VMEM((TPW,), jnp.int32),
        pltpu.VMEM((TPW, D), jnp.float32),
        pltpu.SemaphoreType.DMA,
    ],
)
def _gather(opad_hbm, g_hbm, out_hbm, idx_v, idc_v, rows_v, sem):
    wid = lax.axis_index("s") * NC + lax.axis_index("c")
    base = wid * TPW
    pltpu.sync_copy(g_hbm.at[pl.ds(base, TPW)], idx_v)
    mx = jnp.full((LANES,), 0, jnp.int32)
    for k in range(TPW // LANES):
        v = idx_v[pl.ds(k * LANES, LANES)]
        mx = jnp.maximum(mx, v)
        idc_v[pl.ds(k * LANES, LANES)] = jnp.minimum(v, SLOTS - 1)
    any_ovf = jnp.max(mx) >= SLOTS
    pltpu.async_copy(opad_hbm.at[idc_v], rows_v, sem).wait()

    @pl.when(any_ovf)
    def _():
        for r in range(TPW):
            gv = plsc.load_gather(idx_v, [jnp.full((LANES,), r, jnp.int32)])
            keep = (gv < SLOTS).astype(jnp.float32)

            def col_fix(c2, carry):
                sl = pl.ds(c2 * LANES, LANES)
                rows_v[r, sl] = rows_v[r, sl] * keep
                return carry

            lax.fori_loop(0, D // LANES, col_fix, 0)

    pltpu.sync_copy(rows_v, out_hbm.at[pl.ds(base, TPW)])


def kernel(tokens, expert_ids, w1, w3, w2):
    eids2d = expert_ids.astype(jnp.int32).reshape(RR, WW)
    g = _route(eids2d).reshape(T)
    padded = _scatter(tokens, g)
    opad = _ffn(padded, w1, w3, w2)
    return _gather(opad, g)


# speedup vs baseline: 1.6331x; 1.6331x over previous
import functools

import jax
import jax.numpy as jnp
from jax import lax
from jax.experimental import pallas as pl
from jax.experimental.pallas import tpu as pltpu
from jax.experimental.pallas import tpu_sc as plsc

E = 8
D = 1024
F = 4096
T = 2048
C = 384
SLOTS = E * C
SCRATCH = SLOTS

RR = 256
WW = 8

NC = 2
NS = 16
NW = NC * NS
TPW = T // NW
LANES = 16

FB = 1024
NF = F // FB


def _route_body(eids_ref, g_ref):
    eids = eids_ref[...]
    a = lax.broadcasted_iota(jnp.int32, (RR, RR), 0)
    b = lax.broadcasted_iota(jnp.int32, (RR, RR), 1)
    ltri = (b < a).astype(jnp.float32)

    def incl_cumsum_lanes(m):
        c = m
        for k in (1, 2, 4):
            z = jnp.zeros((RR, k), jnp.float32)
            c = c + jnp.concatenate([z, c[:, : WW - k]], axis=1)
        return c

    withins = []
    rowtots = []
    for e in range(E):
        m = (eids == e).astype(jnp.float32)
        w = incl_cumsum_lanes(m)
        withins.append(w)
        rowtots.append(w[:, WW - 1 : WW])
    rowtot = jnp.concatenate(rowtots, axis=1)
    rowpref = jnp.dot(ltri, rowtot, preferred_element_type=jnp.float32)

    rank = jnp.zeros((RR, WW), jnp.float32)
    for e in range(E):
        m = (eids == e).astype(jnp.float32)
        rank = rank + m * (withins[e] - 1.0 + rowpref[:, e : e + 1])
    rank_i = rank.astype(jnp.int32)
    g = jnp.where(rank_i < C, eids * C + rank_i, SCRATCH)
    g_ref[...] = g.astype(jnp.int32)


def _route(eids2d):
    return pl.pallas_call(
        _route_body,
        out_shape=jax.ShapeDtypeStruct((RR, WW), jnp.int32),
    )(eids2d)


@functools.cache
def _make_scatter():
    mesh = plsc.VectorSubcoreMesh(core_axis_name="c", subcore_axis_name="s")

    @functools.partial(
        pl.kernel,
        mesh=mesh,
        out_type=jax.ShapeDtypeStruct((SLOTS + 1, D), jnp.float32),
        scratch_types=[
            pltpu.VMEM((TPW,), jnp.int32),
            pltpu.VMEM((TPW, D), jnp.float32),
            pltpu.SemaphoreType.DMA,
        ],
    )
    def _scatter(tokens_hbm, g_hbm, padded_hbm, idx_v, rows_v, sem):
        wid = lax.axis_index("s") * NC + lax.axis_index("c")
        base = wid * TPW
        pltpu.sync_copy(g_hbm.at[pl.ds(base, TPW)], idx_v)
        pltpu.sync_copy(tokens_hbm.at[pl.ds(base, TPW)], rows_v)
        pltpu.async_copy(rows_v, padded_hbm.at[idx_v], sem).wait()

    return _scatter


def _ffn_body(x_ref, w1_ref, w3_ref, w2_ref, o_ref):
    f = pl.program_id(1)
    x = x_ref[...].astype(jnp.bfloat16)
    gate = jnp.dot(x, w1_ref[0].astype(jnp.bfloat16),
                   preferred_element_type=jnp.float32)
    up = jnp.dot(x, w3_ref[0].astype(jnp.bfloat16),
                 preferred_element_type=jnp.float32)
    h = (gate * jax.nn.sigmoid(gate)) * up
    acc = jnp.dot(h.astype(jnp.bfloat16), w2_ref[0].astype(jnp.bfloat16),
                  preferred_element_type=jnp.float32)

    @pl.when(f == 0)
    def _():
        o_ref[...] = acc

    @pl.when(f != 0)
    def _():
        o_ref[...] = o_ref[...] + acc


def _ffn(padded, w1, w3, w2):
    return pl.pallas_call(
        _ffn_body,
        grid=(E, NF),
        in_specs=[
            pl.BlockSpec((C, D), lambda e, f: (e, 0)),
            pl.BlockSpec((1, D, FB), lambda e, f: (e, 0, f)),
            pl.BlockSpec((1, D, FB), lambda e, f: (e, 0, f)),
            pl.BlockSpec((1, FB, D), lambda e, f: (e, f, 0)),
        ],
        out_specs=pl.BlockSpec((C, D), lambda e, f: (e, 0)),
        out_shape=jax.ShapeDtypeStruct((SLOTS, D), jnp.float32),
    )(padded, w1, w3, w2)


@functools.cache
def _make_gather():
    mesh = plsc.VectorSubcoreMesh(core_axis_name="c", subcore_axis_name="s")

    @functools.partial(
        pl.kernel,
        mesh=mesh,
        out_type=jax.ShapeDtypeStruct((T, D), jnp.float32),
        scratch_types=[
            pltpu.VMEM((TPW,), jnp.int32),
            pltpu.VMEM((TPW,), jnp.int32),
            pltpu.VMEM((TPW, D), jnp.float32),
            pltpu.SemaphoreType.DMA,
        ],
    )
    def _gather(opad_hbm, g_hbm, out_hbm, idx_v, idc_v, rows_v, sem):
        wid = lax.axis_index("s") * NC + lax.axis_index("c")
        base = wid * TPW
        pltpu.sync_copy(g_hbm.at[pl.ds(base, TPW)], idx_v)
        for k in range(TPW // LANES):
            v = idx_v[pl.ds(k * LANES, LANES)]
            idc_v[pl.ds(k * LANES, LANES)] = jnp.minimum(v, SLOTS - 1)
        any_ovf = jnp.int32(0)
        for k in range(TPW // LANES):
            v = idx_v[pl.ds(k * LANES, LANES)]
            for j in range(LANES):
                any_ovf = any_ovf | (v[j] >= SLOTS).astype(jnp.int32)

        pltpu.async_copy(opad_hbm.at[idc_v], rows_v, sem).wait()

        @pl.when(any_ovf > 0)
        def _():
            for k in range(TPW // LANES):
                v = idx_v[pl.ds(k * LANES, LANES)]
                for j in range(LANES):
                    r = k * LANES + j
                    kv = jnp.full(
                        (LANES,),
                        jnp.where(v[j] >= SLOTS, 0.0, 1.0),
                        jnp.float32,
                    )

                    def col_fix(c2, cc, r=r, kv=kv):
                        sl = pl.ds(c2 * LANES, LANES)
                        rows_v[r, sl] = rows_v[r, sl] * kv
                        return cc

                    lax.fori_loop(0, D // LANES, col_fix, 0)

        pltpu.sync_copy(rows_v, out_hbm.at[pl.ds(base, TPW)])

    return _gather


def kernel(tokens, expert_ids, w1, w3, w2):
    eids2d = expert_ids.astype(jnp.int32).reshape(RR, WW)
    g = _route(eids2d).reshape(T)
    padded = _make_scatter()(tokens, g)
    opad = _ffn(padded, w1, w3, w2)
    return _make_gather()(opad, g)


# speedup vs baseline: 1.6596x; 1.0162x over previous
import functools

import jax
import jax.numpy as jnp
from jax import lax
from jax.experimental import pallas as pl
from jax.experimental.pallas import tpu as pltpu
from jax.experimental.pallas import tpu_sc as plsc

E = 8
D = 1024
F = 4096
T = 2048
C = 384
SLOTS = E * C
SCRATCH = SLOTS

RR = 256
WW = 8

NC = 2
NS = 16
NW = NC * NS
TPW = T // NW
LANES = 16

FB = 2048
NF = F // FB


def _route_body(eids_ref, g_ref):
    eids = eids_ref[...]
    a = lax.broadcasted_iota(jnp.int32, (RR, RR), 0)
    b = lax.broadcasted_iota(jnp.int32, (RR, RR), 1)
    ltri = (b < a).astype(jnp.float32)

    def incl_cumsum_lanes(m):
        c = m
        for k in (1, 2, 4):
            z = jnp.zeros((RR, k), jnp.float32)
            c = c + jnp.concatenate([z, c[:, : WW - k]], axis=1)
        return c

    withins = []
    rowtots = []
    for e in range(E):
        m = (eids == e).astype(jnp.float32)
        w = incl_cumsum_lanes(m)
        withins.append(w)
        rowtots.append(w[:, WW - 1 : WW])
    rowtot = jnp.concatenate(rowtots, axis=1)
    rowpref = jnp.dot(ltri, rowtot, preferred_element_type=jnp.float32)

    rank = jnp.zeros((RR, WW), jnp.float32)
    for e in range(E):
        m = (eids == e).astype(jnp.float32)
        rank = rank + m * (withins[e] - 1.0 + rowpref[:, e : e + 1])
    rank_i = rank.astype(jnp.int32)
    g = jnp.where(rank_i < C, eids * C + rank_i, SCRATCH)
    g_ref[...] = g.astype(jnp.int32)


def _route(eids2d):
    return pl.pallas_call(
        _route_body,
        out_shape=jax.ShapeDtypeStruct((RR, WW), jnp.int32),
    )(eids2d)


@functools.cache
def _make_scatter():
    mesh = plsc.VectorSubcoreMesh(core_axis_name="c", subcore_axis_name="s")

    @functools.partial(
        pl.kernel,
        mesh=mesh,
        out_type=jax.ShapeDtypeStruct((SLOTS + 1, D), jnp.float32),
        scratch_types=[
            pltpu.VMEM((TPW,), jnp.int32),
            pltpu.VMEM((TPW, D), jnp.float32),
            pltpu.SemaphoreType.DMA,
        ],
    )
    def _scatter(tokens_hbm, g_hbm, padded_hbm, idx_v, rows_v, sem):
        wid = lax.axis_index("s") * NC + lax.axis_index("c")
        base = wid * TPW
        pltpu.sync_copy(g_hbm.at[pl.ds(base, TPW)], idx_v)
        pltpu.sync_copy(tokens_hbm.at[pl.ds(base, TPW)], rows_v)
        pltpu.async_copy(rows_v, padded_hbm.at[idx_v], sem).wait()

    return _scatter


def _ffn_body(x_ref, w1_ref, w3_ref, w2_ref, o_ref):
    f = pl.program_id(1)
    x = x_ref[...].astype(jnp.bfloat16)
    gate = jnp.dot(x, w1_ref[0].astype(jnp.bfloat16),
                   preferred_element_type=jnp.float32)
    up = jnp.dot(x, w3_ref[0].astype(jnp.bfloat16),
                 preferred_element_type=jnp.float32)
    h = (gate * jax.nn.sigmoid(gate)) * up
    acc = jnp.dot(h.astype(jnp.bfloat16), w2_ref[0].astype(jnp.bfloat16),
                  preferred_element_type=jnp.float32)

    @pl.when(f == 0)
    def _():
        o_ref[...] = acc

    @pl.when(f != 0)
    def _():
        o_ref[...] = o_ref[...] + acc


def _ffn(padded, w1, w3, w2):
    return pl.pallas_call(
        _ffn_body,
        grid=(E, NF),
        in_specs=[
            pl.BlockSpec((C, D), lambda e, f: (e, 0)),
            pl.BlockSpec((1, D, FB), lambda e, f: (e, 0, f)),
            pl.BlockSpec((1, D, FB), lambda e, f: (e, 0, f)),
            pl.BlockSpec((1, FB, D), lambda e, f: (e, f, 0)),
        ],
        out_specs=pl.BlockSpec((C, D), lambda e, f: (e, 0)),
        out_shape=jax.ShapeDtypeStruct((SLOTS, D), jnp.float32),
    )(padded, w1, w3, w2)


@functools.cache
def _make_gather():
    mesh = plsc.VectorSubcoreMesh(core_axis_name="c", subcore_axis_name="s")

    @functools.partial(
        pl.kernel,
        mesh=mesh,
        out_type=jax.ShapeDtypeStruct((T, D), jnp.float32),
        scratch_types=[
            pltpu.VMEM((TPW,), jnp.int32),
            pltpu.VMEM((TPW,), jnp.int32),
            pltpu.VMEM((TPW, D), jnp.float32),
            pltpu.SemaphoreType.DMA,
        ],
    )
    def _gather(opad_hbm, g_hbm, out_hbm, idx_v, idc_v, rows_v, sem):
        wid = lax.axis_index("s") * NC + lax.axis_index("c")
        base = wid * TPW
        pltpu.sync_copy(g_hbm.at[pl.ds(base, TPW)], idx_v)
        for k in range(TPW // LANES):
            v = idx_v[pl.ds(k * LANES, LANES)]
            idc_v[pl.ds(k * LANES, LANES)] = jnp.minimum(v, SLOTS - 1)
        any_ovf = jnp.int32(0)
        for k in range(TPW // LANES):
            v = idx_v[pl.ds(k * LANES, LANES)]
            for j in range(LANES):
                any_ovf = any_ovf | (v[j] >= SLOTS).astype(jnp.int32)

        pltpu.async_copy(opad_hbm.at[idc_v], rows_v, sem).wait()

        @pl.when(any_ovf > 0)
        def _():
            for k in range(TPW // LANES):
                v = idx_v[pl.ds(k * LANES, LANES)]
                for j in range(LANES):
                    r = k * LANES + j
                    kv = jnp.full(
                        (LANES,),
                        jnp.where(v[j] >= SLOTS, 0.0, 1.0),
                        jnp.float32,
                    )

                    def col_fix(c2, cc, r=r, kv=kv):
                        sl = pl.ds(c2 * LANES, LANES)
                        rows_v[r, sl] = rows_v[r, sl] * kv
                        return cc

                    lax.fori_loop(0, D // LANES, col_fix, 0)

        pltpu.sync_copy(rows_v, out_hbm.at[pl.ds(base, TPW)])

    return _gather


def kernel(tokens, expert_ids, w1, w3, w2):
    eids2d = expert_ids.astype(jnp.int32).reshape(RR, WW)
    g = _route(eids2d).reshape(T)
    padded = _make_scatter()(tokens, g)
    opad = _ffn(padded, w1, w3, w2)
    return _make_gather()(opad, g)


# speedup vs baseline: 1.6879x; 1.0171x over previous
import functools

import jax
import jax.numpy as jnp
from jax import lax
from jax.experimental import pallas as pl
from jax.experimental.pallas import tpu as pltpu
from jax.experimental.pallas import tpu_sc as plsc

E = 8
D = 1024
F = 4096
T = 2048
C = 384
SLOTS = E * C
SCRATCH = SLOTS

RR = 8
WW = 256

NC = 2
NS = 16
NW = NC * NS
TPW = T // NW
LANES = 16

FB = 2048
NF = F // FB


def _route_body(eids_ref, g_ref):
    eids = eids_ref[...]
    a = lax.broadcasted_iota(jnp.int32, (RR, RR), 0)
    b = lax.broadcasted_iota(jnp.int32, (RR, RR), 1)
    ltri = (b < a).astype(jnp.float32)

    def incl_cumsum_lanes(m):
        c = m
        k = 1
        while k < WW:
            z = jnp.zeros((RR, k), jnp.float32)
            c = c + jnp.concatenate([z, c[:, : WW - k]], axis=1)
            k *= 2
        return c

    withins = []
    rowtots = []
    for e in range(E):
        m = (eids == e).astype(jnp.float32)
        w = incl_cumsum_lanes(m)
        withins.append(w)
        rowtots.append(w[:, WW - 1 : WW])
    rowtot = jnp.concatenate(rowtots, axis=1)
    rowpref = jnp.dot(ltri, rowtot, preferred_element_type=jnp.float32)

    rank = jnp.zeros((RR, WW), jnp.float32)
    for e in range(E):
        m = (eids == e).astype(jnp.float32)
        rank = rank + m * (withins[e] - 1.0 + rowpref[:, e : e + 1])
    rank_i = rank.astype(jnp.int32)
    g = jnp.where(rank_i < C, eids * C + rank_i, SCRATCH)
    g_ref[...] = g.astype(jnp.int32)


def _route(eids2d):
    return pl.pallas_call(
        _route_body,
        out_shape=jax.ShapeDtypeStruct((RR, WW), jnp.int32),
    )(eids2d)


@functools.cache
def _make_scatter():
    mesh = plsc.VectorSubcoreMesh(core_axis_name="c", subcore_axis_name="s")

    @functools.partial(
        pl.kernel,
        mesh=mesh,
        out_type=jax.ShapeDtypeStruct((SLOTS + 1, D), jnp.float32),
        scratch_types=[
            pltpu.VMEM((TPW,), jnp.int32),
            pltpu.VMEM((TPW, D), jnp.float32),
            pltpu.SemaphoreType.DMA,
        ],
    )
    def _scatter(tokens_hbm, g_hbm, padded_hbm, idx_v, rows_v, sem):
        wid = lax.axis_index("s") * NC + lax.axis_index("c")
        base = wid * TPW
        pltpu.sync_copy(
            g_hbm.at[wid // (WW // TPW), pl.ds((wid % (WW // TPW)) * TPW, TPW)],
            idx_v)
        pltpu.sync_copy(tokens_hbm.at[pl.ds(base, TPW)], rows_v)
        pltpu.async_copy(rows_v, padded_hbm.at[idx_v], sem).wait()

    return _scatter


def _ffn_body(x_ref, w1_ref, w3_ref, w2_ref, o_ref):
    f = pl.program_id(1)
    x = x_ref[...].astype(jnp.bfloat16)
    gate = jnp.dot(x, w1_ref[0].astype(jnp.bfloat16),
                   preferred_element_type=jnp.float32)
    up = jnp.dot(x, w3_ref[0].astype(jnp.bfloat16),
                 preferred_element_type=jnp.float32)
    h = (gate * jax.nn.sigmoid(gate)) * up
    acc = jnp.dot(h.astype(jnp.bfloat16), w2_ref[0].astype(jnp.bfloat16),
                  preferred_element_type=jnp.float32)

    @pl.when(f == 0)
    def _():
        o_ref[...] = acc

    @pl.when(f != 0)
    def _():
        o_ref[...] = o_ref[...] + acc


def _ffn(padded, w1, w3, w2):
    return pl.pallas_call(
        _ffn_body,
        grid=(E, NF),
        in_specs=[
            pl.BlockSpec((C, D), lambda e, f: (e, 0)),
            pl.BlockSpec((1, D, FB), lambda e, f: (e, 0, f)),
            pl.BlockSpec((1, D, FB), lambda e, f: (e, 0, f)),
            pl.BlockSpec((1, FB, D), lambda e, f: (e, f, 0)),
        ],
        out_specs=pl.BlockSpec((C, D), lambda e, f: (e, 0)),
        out_shape=jax.ShapeDtypeStruct((SLOTS, D), jnp.float32),
    )(padded, w1, w3, w2)


@functools.cache
def _make_gather():
    mesh = plsc.VectorSubcoreMesh(core_axis_name="c", subcore_axis_name="s")

    @functools.partial(
        pl.kernel,
        mesh=mesh,
        out_type=jax.ShapeDtypeStruct((T, D), jnp.float32),
        scratch_types=[
            pltpu.VMEM((TPW,), jnp.int32),
            pltpu.VMEM((TPW,), jnp.int32),
            pltpu.VMEM((TPW, D), jnp.float32),
            pltpu.SemaphoreType.DMA,
        ],
    )
    def _gather(opad_hbm, g_hbm, out_hbm, idx_v, idc_v, rows_v, sem):
        wid = lax.axis_index("s") * NC + lax.axis_index("c")
        base = wid * TPW
        pltpu.sync_copy(
            g_hbm.at[wid // (WW // TPW), pl.ds((wid % (WW // TPW)) * TPW, TPW)],
            idx_v)
        for k in range(TPW // LANES):
            v = idx_v[pl.ds(k * LANES, LANES)]
            idc_v[pl.ds(k * LANES, LANES)] = jnp.minimum(v, SLOTS - 1)
        any_ovf = jnp.int32(0)
        for k in range(TPW // LANES):
            v = idx_v[pl.ds(k * LANES, LANES)]
            for j in range(LANES):
                any_ovf = any_ovf | (v[j] >= SLOTS).astype(jnp.int32)

        pltpu.async_copy(opad_hbm.at[idc_v], rows_v, sem).wait()

        @pl.when(any_ovf > 0)
        def _():
            for k in range(TPW // LANES):
                v = idx_v[pl.ds(k * LANES, LANES)]
                for j in range(LANES):
                    r = k * LANES + j
                    kv = jnp.full(
                        (LANES,),
                        jnp.where(v[j] >= SLOTS, 0.0, 1.0),
                        jnp.float32,
                    )

                    def col_fix(c2, cc, r=r, kv=kv):
                        sl = pl.ds(c2 * LANES, LANES)
                        rows_v[r, sl] = rows_v[r, sl] * kv
                        return cc

                    lax.fori_loop(0, D // LANES, col_fix, 0)

        pltpu.sync_copy(rows_v, out_hbm.at[pl.ds(base, TPW)])

    return _gather


def kernel(tokens, expert_ids, w1, w3, w2):
    eids2d = expert_ids.astype(jnp.int32).reshape(RR, WW)
    g = _route(eids2d)
    padded = _make_scatter()(tokens, g)
    opad = _ffn(padded, w1, w3, w2)
    return _make_gather()(opad, g)
